# baseline (speedup 1.0000x reference)
import functools

import jax
import jax.numpy as jnp
from jax import lax
from jax.experimental import pallas as pl
from jax.experimental.pallas import tpu as pltpu
from jax.experimental.pallas import tpu_sc as plsc

_NC = 2
_NS = 16
_L = 16
_NBUF = 4


@functools.lru_cache(maxsize=None)
def _build(B, S, V, D):
  NW = _NC * _NS
  assert B % NW == 0
  BPW = B // NW
  NCHUNK = S
  assert NCHUNK % _NBUF == 0 and NCHUNK // _NBUF >= 3
  assert BPW % _L == 0 and BPW <= 128 and BPW % 8 == 0
  assert D % _L == 0

  mesh = plsc.VectorSubcoreMesh(core_axis_name="c", subcore_axis_name="s")

  @functools.partial(
      pl.kernel,
      mesh=mesh,
      compiler_params=pltpu.CompilerParams(use_tc_tiling_on_sc=False),
      out_type=jax.ShapeDtypeStruct((B, S, D), jnp.float32),
      scratch_types=[
          pltpu.VMEM((S, BPW), jnp.float32),
          pltpu.VMEM((_NBUF, BPW), jnp.int32),
          pltpu.VMEM((S, D), jnp.float32),
          pltpu.VMEM((_NBUF, BPW, D), jnp.float32),
          pltpu.SemaphoreType.DMA,
          pltpu.SemaphoreType.DMA,
      ],
  )
  def emb(seq_hbm, tok_hbm, pos_hbm, out_hbm, idxf_v, idx_v, pos_v, rows_v,
          gsem, osem):
    wid = lax.axis_index("s") * _NC + lax.axis_index("c")
    b0 = wid * BPW
    pltpu.sync_copy(pos_hbm, pos_v)
    pltpu.sync_copy(seq_hbm.at[:, pl.ds(b0, BPW)], idxf_v)

    def issue_gather(g, buf):
      for j in range(BPW // _L):
        idx_v[buf, pl.ds(j * _L, _L)] = jax.lax.bitcast_convert_type(
            idxf_v[g, pl.ds(j * _L, _L)], jnp.int32)
      pltpu.async_copy(tok_hbm.at[idx_v.at[buf]], rows_v.at[buf], gsem)

    def wait_gather(buf):
      pltpu.make_async_copy(
          tok_hbm.at[pl.ds(0, BPW), :], rows_v.at[buf], gsem).wait()

    def issue_out(g, buf):
      pltpu.async_copy(rows_v.at[buf], out_hbm.at[pl.ds(b0, BPW), g, :],
                       osem)

    def wait_out(buf):
      pltpu.make_async_copy(rows_v.at[buf],
                            out_hbm.at[pl.ds(b0, BPW), 0, :], osem).wait()

    def add_pos(g, buf):
      def b_body(bl, carry):
        for j in range(D // _L):
          pv = pos_v[g, pl.ds(j * _L, _L)]
          plsc.addupdate(rows_v.at[buf, bl, pl.ds(j * _L, _L)], pv)
        return carry

      lax.fori_loop(0, BPW, b_body, 0)

    def slot(g, b, *, first=False, last=False):
      wait_gather(b)
      if not first:
        wait_out((b + 3) % _NBUF)
      if not last:
        issue_gather(g + 3, (b + 3) % _NBUF)
      add_pos(g, b)
      issue_out(g, b)

    for g in range(3):
      issue_gather(g, g)

    slot(0, 0, first=True)
    for b in range(1, _NBUF):
      slot(b, b)

    def turn(g4, carry):
      for b in range(_NBUF):
        slot(g4 * _NBUF + b, b)
      return carry

    lax.fori_loop(1, NCHUNK // _NBUF - 1, turn, 0)

    gl = NCHUNK - _NBUF
    slot(gl, 0)
    for b in range(1, _NBUF):
      slot(gl + b, b, last=True)
    wait_out(_NBUF - 1)

  return emb


def kernel(seq, token_table, pos_table):
  B, S = seq.shape
  V, D = token_table.shape
  emb = _build(B, S, V, D)
  seq_bits = jax.lax.bitcast_convert_type(seq, jnp.float32).T
  return emb(seq_bits, token_table, pos_table)


# speedup vs baseline: 1.0194x; 1.0194x over previous
import functools

import jax
import jax.numpy as jnp
from jax import lax
from jax.experimental import pallas as pl
from jax.experimental.pallas import tpu as pltpu
from jax.experimental.pallas import tpu_sc as plsc

_NC = 2
_NS = 16
_L = 16
_NBUF = 4


@functools.lru_cache(maxsize=None)
def _build(B, S, V, D):
  NW = _NC * _NS
  assert B % NW == 0
  BPW = B // NW
  NCHUNK = S
  assert NCHUNK % _NBUF == 0 and NCHUNK // _NBUF >= 3
  assert BPW % _L == 0 and BPW <= 128 and BPW % 8 == 0
  assert D % _L == 0

  mesh = plsc.VectorSubcoreMesh(core_axis_name="c", subcore_axis_name="s")

  @functools.partial(
      pl.kernel,
      mesh=mesh,
      compiler_params=pltpu.CompilerParams(use_tc_tiling_on_sc=False),
      out_type=jax.ShapeDtypeStruct((S, B, D), jnp.float32),
      scratch_types=[
          pltpu.VMEM((S, BPW), jnp.float32),
          pltpu.VMEM((_NBUF, BPW), jnp.int32),
          pltpu.VMEM((S, D), jnp.float32),
          pltpu.VMEM((_NBUF, BPW, D), jnp.float32),
          pltpu.SemaphoreType.DMA,
          pltpu.SemaphoreType.DMA,
      ],
  )
  def emb(seq_hbm, tok_hbm, pos_hbm, out_hbm, idxf_v, idx_v, pos_v, rows_v,
          gsem, osem):
    wid = lax.axis_index("s") * _NC + lax.axis_index("c")
    b0 = wid * BPW
    pltpu.sync_copy(pos_hbm, pos_v)
    pltpu.sync_copy(seq_hbm.at[:, pl.ds(b0, BPW)], idxf_v)

    def issue_gather(g, buf):
      for j in range(BPW // _L):
        idx_v[buf, pl.ds(j * _L, _L)] = jax.lax.bitcast_convert_type(
            idxf_v[g, pl.ds(j * _L, _L)], jnp.int32)
      pltpu.async_copy(tok_hbm.at[idx_v.at[buf]], rows_v.at[buf], gsem)

    def wait_gather(buf):
      pltpu.make_async_copy(
          tok_hbm.at[pl.ds(0, BPW), :], rows_v.at[buf], gsem).wait()

    def issue_out(g, buf):
      pltpu.async_copy(rows_v.at[buf], out_hbm.at[g, pl.ds(b0, BPW), :],
                       osem)

    def wait_out(buf):
      pltpu.make_async_copy(rows_v.at[buf],
                            out_hbm.at[0, pl.ds(b0, BPW), :], osem).wait()

    def add_pos(g, buf):
      def b_body(bl, carry):
        for j in range(D // _L):
          pv = pos_v[g, pl.ds(j * _L, _L)]
          plsc.addupdate(rows_v.at[buf, bl, pl.ds(j * _L, _L)], pv)
        return carry

      lax.fori_loop(0, BPW, b_body, 0)

    def slot(g, b, *, first=False, last=False):
      wait_gather(b)
      if not first:
        wait_out((b + 3) % _NBUF)
      if not last:
        issue_gather(g + 3, (b + 3) % _NBUF)
      add_pos(g, b)
      issue_out(g, b)

    for g in range(3):
      issue_gather(g, g)

    slot(0, 0, first=True)
    for b in range(1, _NBUF):
      slot(b, b)

    def turn(g4, carry):
      for b in range(_NBUF):
        slot(g4 * _NBUF + b, b)
      return carry

    lax.fori_loop(1, NCHUNK // _NBUF - 1, turn, 0)

    gl = NCHUNK - _NBUF
    slot(gl, 0)
    for b in range(1, _NBUF):
      slot(gl + b, b, last=True)
    wait_out(_NBUF - 1)

  return emb


def kernel(seq, token_table, pos_table):
  B, S = seq.shape
  V, D = token_table.shape
  emb = _build(B, S, V, D)
  seq_bits = jax.lax.bitcast_convert_type(seq, jnp.float32).T
  out = emb(seq_bits, token_table, pos_table)
  return jnp.transpose(out, (1, 0, 2))
